# Initial kernel scaffold; baseline (speedup 1.0000x reference)
import jax
import jax.numpy as jnp
from jax.experimental import pallas as pl


def kernel(x, Wr, br, Wa, ba, Wt, bt):
    raise NotImplementedError("write your pallas kernel here")


# speedup vs baseline: 3.3212x; 3.3212x over previous
import functools

import jax
import jax.numpy as jnp
from jax import lax
from jax.experimental import pallas as pl
from jax.experimental.pallas import tpu as pltpu
from jax.experimental.pallas import tpu_sc as plsc


def _router_body(x_ref, wr_ref, br_ref, rw_ref):
    rw_ref[...] = (
        jnp.dot(x_ref[...], wr_ref[...], preferred_element_type=jnp.float32)
        + br_ref[0, 0]
    )


def _router(x_flat, Wr, br, n_rows, d, blk):
    return pl.pallas_call(
        _router_body,
        grid=(n_rows // blk,),
        in_specs=[
            pl.BlockSpec((blk, d), lambda i: (i, 0)),
            pl.BlockSpec((d, 1), lambda i: (0, 0)),
            pl.BlockSpec(memory_space=pltpu.SMEM),
        ],
        out_specs=pl.BlockSpec((blk, 1), lambda i: (i, 0)),
        out_shape=jax.ShapeDtypeStruct((n_rows, 1), jnp.float32),
        name="mod_router",
    )(x_flat, Wr, br.reshape(1, 1))


def _make_topk(b_count, s_len, k):
    mesh = plsc.VectorSubcoreMesh(core_axis_name="c", subcore_axis_name="s")
    batches_per_core = b_count // 2
    n_vregs = s_len // 16

    @functools.partial(
        pl.kernel,
        out_type=[
            jax.ShapeDtypeStruct((b_count * k,), jnp.int32),
            jax.ShapeDtypeStruct((b_count * k,), jnp.float32),
        ],
        mesh=mesh,
        scratch_types=[
            pltpu.VMEM((s_len,), jnp.float32),
            pltpu.VMEM((s_len,), jnp.uint32),
            pltpu.VMEM((k,), jnp.int32),
            pltpu.VMEM((k,), jnp.float32),
        ],
        compiler_params=pltpu.CompilerParams(needs_layout_passes=False),
    )
    def topk_kernel(rw_hbm, topi_hbm, selw_hbm, vals_v, keys_v, ti_v, tw_v):
        c = lax.axis_index("c")
        s = lax.axis_index("s")
        b = c * batches_per_core + s

        @pl.when(s < batches_per_core)
        def _():
            pltpu.sync_copy(rw_hbm.at[b], vals_v)

            def key_body(j, _):
                base = j * 128
                for u in range(8):
                    off = base + u * 16
                    bu = plsc.bitcast(vals_v[pl.ds(off, 16)], jnp.uint32)
                    keys_v[pl.ds(off, 16)] = jnp.where(
                        bu >= jnp.uint32(0x80000000),
                        ~bu,
                        bu | jnp.uint32(0x80000000),
                    )
                return 0

            lax.fori_loop(0, n_vregs // 8, key_body, 0)

            def count_cmp(thresh, strict):
                def body(j, acc):
                    base = j * 128
                    for u in range(8):
                        kv = keys_v[pl.ds(base + u * 16, 16)]
                        hit = kv > thresh if strict else kv >= thresh
                        acc = acc + hit.astype(jnp.int32)
                    return acc

                acc = lax.fori_loop(
                    0, n_vregs // 8, body, jnp.zeros((16,), jnp.int32)
                )
                return jnp.sum(acc)

            t = jnp.uint32(0)
            for bit in range(31, -1, -1):
                cand = t | jnp.uint32(1 << bit)
                t = jnp.where(count_cmp(cand, False) >= k, cand, t)

            need = k - count_cmp(t, True)
            lane = lax.iota(jnp.int32, 16)

            def comp_body(j, carry):
                off, eqs = carry
                base = j * 16
                kv = keys_v[pl.ds(base, 16)]
                vv = vals_v[pl.ds(base, 16)]
                gt = kv > t
                eq = kv == t
                eqc = plsc.cumsum(eq.astype(jnp.int32))
                sel = jnp.logical_or(
                    gt, jnp.logical_and(eq, (eqs + eqc) <= need)
                )
                sc = plsc.cumsum(sel.astype(jnp.int32))
                dest = off + sc - 1
                gidx = base + lane + b * s_len
                plsc.store_scatter(ti_v, [dest], gidx, mask=sel)
                plsc.store_scatter(tw_v, [dest], vv, mask=sel)
                return (
                    off + jnp.sum(sel.astype(jnp.int32)),
                    eqs + jnp.sum(eq.astype(jnp.int32)),
                )

            lax.fori_loop(0, n_vregs, comp_body, (jnp.int32(0), jnp.int32(0)))

            pltpu.sync_copy(ti_v, topi_hbm.at[pl.ds(b * k, k)])
            pltpu.sync_copy(tw_v, selw_hbm.at[pl.ds(b * k, k)])

    return topk_kernel


def _make_gather(n_sel, d):
    mesh = plsc.VectorSubcoreMesh(core_axis_name="c", subcore_axis_name="s")
    rows_per_tile = n_sel // 32

    @functools.partial(
        pl.kernel,
        out_type=jax.ShapeDtypeStruct((n_sel, d), jnp.float32),
        mesh=mesh,
        scratch_types=[
            pltpu.VMEM((rows_per_tile,), jnp.int32),
            pltpu.VMEM((rows_per_tile, d), jnp.float32),
            pltpu.SemaphoreType.DMA,
        ],
    )
    def gather_kernel(x_hbm, topi_hbm, xsel_hbm, idx_v, rows_v, sem):
        c = lax.axis_index("c")
        s = lax.axis_index("s")
        base = (s * 2 + c) * rows_per_tile
        pltpu.sync_copy(topi_hbm.at[pl.ds(base, rows_per_tile)], idx_v)
        pltpu.async_copy(x_hbm.at[idx_v], rows_v, sem).wait()
        pltpu.sync_copy(rows_v, xsel_hbm.at[pl.ds(base, rows_per_tile)])

    return gather_kernel


def _block_body(xs_ref, wt_ref, bt_ref, sw_ref, y_ref):
    y_ref[...] = (
        jnp.dot(xs_ref[...], wt_ref[...], preferred_element_type=jnp.float32)
        + bt_ref[...]
    ) * sw_ref[...]


def _block_mm(xsel, Wt, bt, selw, n_sel, d, blk):
    return pl.pallas_call(
        _block_body,
        grid=(n_sel // blk,),
        in_specs=[
            pl.BlockSpec((blk, d), lambda i: (i, 0)),
            pl.BlockSpec((d, d), lambda i: (0, 0)),
            pl.BlockSpec((1, d), lambda i: (0, 0)),
            pl.BlockSpec((blk, 1), lambda i: (i, 0)),
        ],
        out_specs=pl.BlockSpec((blk, d), lambda i: (i, 0)),
        out_shape=jax.ShapeDtypeStruct((n_sel, d), jnp.float32),
        name="mod_block_mm",
    )(xsel, Wt, bt.reshape(1, d), selw.reshape(n_sel, 1))


def _make_copy_scatter(n_rows, d, n_sel, chunk):
    mesh = plsc.VectorSubcoreMesh(core_axis_name="c", subcore_axis_name="s")
    rows_per_tile = n_rows // 32
    sel_per_tile = n_sel // 32
    n_chunks = rows_per_tile // chunk
    sel_chunks = sel_per_tile // chunk

    @functools.partial(
        pl.kernel,
        out_type=jax.ShapeDtypeStruct((n_rows, d), jnp.float32),
        mesh=mesh,
        scratch_types=[
            pltpu.VMEM((chunk, d), jnp.float32),
            pltpu.VMEM((chunk,), jnp.int32),
            pltpu.SemaphoreType.DMA,
        ],
    )
    def copy_scatter_kernel(x_hbm, topi_hbm, y_hbm, out_hbm, buf, idx_v, sem):
        c = lax.axis_index("c")
        s = lax.axis_index("s")
        row0 = (c * 16 + s) * rows_per_tile

        def copy_body(j, _):
            r = row0 + j * chunk
            pltpu.sync_copy(x_hbm.at[pl.ds(r, chunk)], buf)
            pltpu.sync_copy(buf, out_hbm.at[pl.ds(r, chunk)])
            return 0

        lax.fori_loop(0, n_chunks, copy_body, 0)

        plsc.subcore_barrier()

        sbase = (c * 16 + s) * sel_per_tile
        for ch in range(sel_chunks):
            o = sbase + ch * chunk
            pltpu.sync_copy(topi_hbm.at[pl.ds(o, chunk)], idx_v)
            pltpu.sync_copy(y_hbm.at[pl.ds(o, chunk)], buf)
            pltpu.async_copy(buf, out_hbm.at[idx_v], sem).wait()

    return copy_scatter_kernel


def kernel(x, Wr, br, Wa, ba, Wt, bt):
    B, S, D = x.shape
    k = S // 8
    n_rows = B * S
    n_sel = B * k
    assert B == 4 and S % 16 == 0 and D % 16 == 0 and n_sel % 32 == 0

    x_flat = x.reshape(n_rows, D)
    rw = _router(x_flat, Wr, br, n_rows, D, 512)

    topi, selw = _make_topk(B, S, k)(rw.reshape(B, S))
    xsel = _make_gather(n_sel, D)(x_flat, topi)
    y = _block_mm(xsel, Wt, bt, selw, n_sel, D, 512)
    out = _make_copy_scatter(n_rows, D, n_sel, 64)(x_flat, topi, y)
    return out.reshape(B, S, D)


# speedup vs baseline: 4.4195x; 1.3307x over previous
import functools

import jax
import jax.numpy as jnp
from jax import lax
from jax.experimental import pallas as pl
from jax.experimental.pallas import tpu as pltpu
from jax.experimental.pallas import tpu_sc as plsc


def _router_body(x_ref, wr_ref, br_ref, rw_ref, out_ref):
    xb = x_ref[...]
    out_ref[...] = xb
    rw_ref[...] = (
        jnp.dot(xb, wr_ref[...], preferred_element_type=jnp.float32)
        + br_ref[0, 0]
    )


def _router(x_flat, Wr, br, n_rows, d, blk):
    return pl.pallas_call(
        _router_body,
        grid=(n_rows // blk,),
        in_specs=[
            pl.BlockSpec((blk, d), lambda i: (i, 0)),
            pl.BlockSpec((d, 1), lambda i: (0, 0)),
            pl.BlockSpec(memory_space=pltpu.SMEM),
        ],
        out_specs=[
            pl.BlockSpec((blk, 1), lambda i: (i, 0)),
            pl.BlockSpec((blk, d), lambda i: (i, 0)),
        ],
        out_shape=[
            jax.ShapeDtypeStruct((n_rows, 1), jnp.float32),
            jax.ShapeDtypeStruct((n_rows, d), jnp.float32),
        ],
        name="mod_router",
    )(x_flat, Wr, br.reshape(1, 1))


def _make_topk(b_count, s_len, k):
    mesh = plsc.VectorSubcoreMesh(core_axis_name="c", subcore_axis_name="s")
    batches_per_core = b_count // 2
    n_vregs = s_len // 16

    @functools.partial(
        pl.kernel,
        out_type=[
            jax.ShapeDtypeStruct((b_count * k,), jnp.int32),
            jax.ShapeDtypeStruct((b_count * k,), jnp.float32),
        ],
        mesh=mesh,
        scratch_types=[
            pltpu.VMEM((s_len,), jnp.float32),
            pltpu.VMEM((s_len,), jnp.uint32),
            pltpu.VMEM((k,), jnp.int32),
            pltpu.VMEM((k,), jnp.float32),
        ],
        compiler_params=pltpu.CompilerParams(needs_layout_passes=False),
    )
    def topk_kernel(rw_hbm, topi_hbm, selw_hbm, vals_v, keys_v, ti_v, tw_v):
        c = lax.axis_index("c")
        s = lax.axis_index("s")
        b = c * batches_per_core + s

        @pl.when(s < batches_per_core)
        def _():
            pltpu.sync_copy(rw_hbm.at[b], vals_v)

            def key_body(j, _):
                base = j * 128
                for u in range(8):
                    off = base + u * 16
                    bu = plsc.bitcast(vals_v[pl.ds(off, 16)], jnp.uint32)
                    keys_v[pl.ds(off, 16)] = jnp.where(
                        bu >= jnp.uint32(0x80000000),
                        ~bu,
                        bu | jnp.uint32(0x80000000),
                    )
                return 0

            lax.fori_loop(0, n_vregs // 8, key_body, 0)

            def count_cmp(thresh, strict):
                def body(j, acc):
                    base = j * 128
                    for u in range(8):
                        kv = keys_v[pl.ds(base + u * 16, 16)]
                        hit = kv > thresh if strict else kv >= thresh
                        acc = acc + hit.astype(jnp.int32)
                    return acc

                acc = lax.fori_loop(
                    0, n_vregs // 8, body, jnp.zeros((16,), jnp.int32)
                )
                return jnp.sum(acc)

            t = jnp.uint32(0)
            for bit in range(31, -1, -1):
                cand = t | jnp.uint32(1 << bit)
                t = jnp.where(count_cmp(cand, False) >= k, cand, t)

            need = k - count_cmp(t, True)
            lane = lax.iota(jnp.int32, 16)

            def comp_body(j, carry):
                off, eqs = carry
                base = j * 16
                kv = keys_v[pl.ds(base, 16)]
                vv = vals_v[pl.ds(base, 16)]
                gt = kv > t
                eq = kv == t
                eqc = plsc.cumsum(eq.astype(jnp.int32))
                sel = jnp.logical_or(
                    gt, jnp.logical_and(eq, (eqs + eqc) <= need)
                )
                sc = plsc.cumsum(sel.astype(jnp.int32))
                dest = off + sc - 1
                gidx = base + lane + b * s_len
                plsc.store_scatter(ti_v, [dest], gidx, mask=sel)
                plsc.store_scatter(tw_v, [dest], vv, mask=sel)
                return (
                    off + jnp.sum(sel.astype(jnp.int32)),
                    eqs + jnp.sum(eq.astype(jnp.int32)),
                )

            lax.fori_loop(0, n_vregs, comp_body, (jnp.int32(0), jnp.int32(0)))

            pltpu.sync_copy(ti_v, topi_hbm.at[pl.ds(b * k, k)])
            pltpu.sync_copy(tw_v, selw_hbm.at[pl.ds(b * k, k)])

    return topk_kernel


def _make_gather(n_sel, d):
    mesh = plsc.VectorSubcoreMesh(core_axis_name="c", subcore_axis_name="s")
    rows_per_tile = n_sel // 32

    @functools.partial(
        pl.kernel,
        out_type=jax.ShapeDtypeStruct((n_sel, d), jnp.float32),
        mesh=mesh,
        scratch_types=[
            pltpu.VMEM((rows_per_tile,), jnp.int32),
            pltpu.VMEM((rows_per_tile, d), jnp.float32),
            pltpu.SemaphoreType.DMA,
        ],
    )
    def gather_kernel(x_hbm, topi_hbm, xsel_hbm, idx_v, rows_v, sem):
        c = lax.axis_index("c")
        s = lax.axis_index("s")
        base = (s * 2 + c) * rows_per_tile
        pltpu.sync_copy(topi_hbm.at[pl.ds(base, rows_per_tile)], idx_v)
        pltpu.async_copy(x_hbm.at[idx_v], rows_v, sem).wait()
        pltpu.sync_copy(rows_v, xsel_hbm.at[pl.ds(base, rows_per_tile)])

    return gather_kernel


def _block_body(xs_ref, wt_ref, bt_ref, sw_ref, y_ref):
    y_ref[...] = (
        jnp.dot(xs_ref[...], wt_ref[...], preferred_element_type=jnp.float32)
        + bt_ref[...]
    ) * sw_ref[...]


def _block_mm(xsel, Wt, bt, selw, n_sel, d, blk):
    return pl.pallas_call(
        _block_body,
        grid=(n_sel // blk,),
        in_specs=[
            pl.BlockSpec((blk, d), lambda i: (i, 0)),
            pl.BlockSpec((d, d), lambda i: (0, 0)),
            pl.BlockSpec((1, d), lambda i: (0, 0)),
            pl.BlockSpec((blk, 1), lambda i: (i, 0)),
        ],
        out_specs=pl.BlockSpec((blk, d), lambda i: (i, 0)),
        out_shape=jax.ShapeDtypeStruct((n_sel, d), jnp.float32),
        name="mod_block_mm",
    )(xsel, Wt, bt.reshape(1, d), selw.reshape(n_sel, 1))


def _make_scatter(n_rows, d, n_sel):
    mesh = plsc.VectorSubcoreMesh(core_axis_name="c", subcore_axis_name="s")
    sel_per_tile = n_sel // 32

    @functools.partial(
        pl.kernel,
        out_type=(),
        mesh=mesh,
        scratch_types=[
            pltpu.VMEM((sel_per_tile, d), jnp.float32),
            pltpu.VMEM((sel_per_tile,), jnp.int32),
            pltpu.SemaphoreType.DMA,
        ],
    )
    def scatter_kernel(out_ref, topi_hbm, y_hbm, buf, idx_v, sem):
        c = lax.axis_index("c")
        s = lax.axis_index("s")
        o = (s * 2 + c) * sel_per_tile
        pltpu.sync_copy(topi_hbm.at[pl.ds(o, sel_per_tile)], idx_v)
        pltpu.sync_copy(y_hbm.at[pl.ds(o, sel_per_tile)], buf)
        pltpu.async_copy(buf, out_ref.at[idx_v], sem).wait()

    return scatter_kernel


def kernel(x, Wr, br, Wa, ba, Wt, bt):
    B, S, D = x.shape
    k = S // 8
    n_rows = B * S
    n_sel = B * k
    assert B == 4 and S % 16 == 0 and D % 16 == 0 and n_sel % 32 == 0

    x_flat = x.reshape(n_rows, D)
    rw, out0 = _router(x_flat, Wr, br, n_rows, D, 512)

    topi, selw = _make_topk(B, S, k)(rw.reshape(B, S))
    xsel = _make_gather(n_sel, D)(x_flat, topi)
    y = _block_mm(xsel, Wt, bt, selw, n_sel, D, 512)
    out_ref = jax.new_ref(out0)
    _make_scatter(n_rows, D, n_sel)(out_ref, topi, y)
    return jax.freeze(out_ref).reshape(B, S, D)


# speedup vs baseline: 4.8687x; 1.1016x over previous
import functools

import jax
import jax.numpy as jnp
from jax import lax
from jax.experimental import pallas as pl
from jax.experimental.pallas import tpu as pltpu
from jax.experimental.pallas import tpu_sc as plsc


def _router_body(x_ref, wr_ref, br_ref, rw_ref, out_ref):
    xb = x_ref[...]
    out_ref[...] = xb
    rw_ref[...] = (
        jnp.dot(xb, wr_ref[...], preferred_element_type=jnp.float32)
        + br_ref[0, 0]
    )


def _router(x_flat, Wr, br, n_rows, d, blk):
    return pl.pallas_call(
        _router_body,
        grid=(n_rows // blk,),
        in_specs=[
            pl.BlockSpec((blk, d), lambda i: (i, 0)),
            pl.BlockSpec((d, 1), lambda i: (0, 0)),
            pl.BlockSpec(memory_space=pltpu.SMEM),
        ],
        out_specs=[
            pl.BlockSpec((blk, 1), lambda i: (i, 0)),
            pl.BlockSpec((blk, d), lambda i: (i, 0)),
        ],
        out_shape=[
            jax.ShapeDtypeStruct((n_rows, 1), jnp.float32),
            jax.ShapeDtypeStruct((n_rows, d), jnp.float32),
        ],
        name="mod_router",
    )(x_flat, Wr, br.reshape(1, 1))


def _make_topk(b_count, s_len, k):
    mesh = plsc.VectorSubcoreMesh(core_axis_name="c", subcore_axis_name="s")
    batches_per_core = b_count // 2
    n_vregs = s_len // 16

    @functools.partial(
        pl.kernel,
        out_type=[
            jax.ShapeDtypeStruct((b_count * k,), jnp.int32),
            jax.ShapeDtypeStruct((b_count * k,), jnp.float32),
        ],
        mesh=mesh,
        scratch_types=[
            pltpu.VMEM((s_len,), jnp.float32),
            pltpu.VMEM((s_len,), jnp.uint32),
            pltpu.VMEM((k,), jnp.int32),
            pltpu.VMEM((k,), jnp.float32),
        ],
        compiler_params=pltpu.CompilerParams(needs_layout_passes=False),
    )
    def topk_kernel(rw_hbm, topi_hbm, selw_hbm, vals_v, keys_v, ti_v, tw_v):
        c = lax.axis_index("c")
        s = lax.axis_index("s")
        b = c * batches_per_core + s

        @pl.when(s < batches_per_core)
        def _():
            pltpu.sync_copy(rw_hbm.at[b], vals_v)

            def key_body(j, _):
                base = j * 128
                for u in range(8):
                    off = base + u * 16
                    bu = plsc.bitcast(vals_v[pl.ds(off, 16)], jnp.uint32)
                    keys_v[pl.ds(off, 16)] = jnp.where(
                        bu >= jnp.uint32(0x80000000),
                        ~bu,
                        bu | jnp.uint32(0x80000000),
                    )
                return 0

            lax.fori_loop(0, n_vregs // 8, key_body, 0)

            def count_cmp(thresh, strict):
                def body(j, acc):
                    base = j * 128
                    for u in range(8):
                        kv = keys_v[pl.ds(base + u * 16, 16)]
                        hit = kv > thresh if strict else kv >= thresh
                        acc = acc + hit.astype(jnp.int32)
                    return acc

                acc = lax.fori_loop(
                    0, n_vregs // 8, body, jnp.zeros((16,), jnp.int32)
                )
                return jnp.sum(acc)

            t = jnp.uint32(0)
            for bit in range(31, -1, -1):
                cand = t | jnp.uint32(1 << bit)
                t = jnp.where(count_cmp(cand, False) >= k, cand, t)

            need = k - count_cmp(t, True)
            lane = lax.iota(jnp.int32, 16)

            def comp_body(j, carry):
                off, eqs = carry
                base = j * 16
                kv = keys_v[pl.ds(base, 16)]
                vv = vals_v[pl.ds(base, 16)]
                gt = kv > t
                eq = kv == t
                eqc = plsc.cumsum(eq.astype(jnp.int32))
                sel = jnp.logical_or(
                    gt, jnp.logical_and(eq, (eqs + eqc) <= need)
                )
                sc = plsc.cumsum(sel.astype(jnp.int32))
                dest = off + sc - 1
                gidx = base + lane + b * s_len
                plsc.store_scatter(ti_v, [dest], gidx, mask=sel)
                plsc.store_scatter(tw_v, [dest], vv, mask=sel)
                return (
                    off + jnp.sum(sel.astype(jnp.int32)),
                    eqs + jnp.sum(eq.astype(jnp.int32)),
                )

            lax.fori_loop(0, n_vregs, comp_body, (jnp.int32(0), jnp.int32(0)))

            pltpu.sync_copy(ti_v, topi_hbm.at[pl.ds(b * k, k)])
            pltpu.sync_copy(tw_v, selw_hbm.at[pl.ds(b * k, k)])

    return topk_kernel


def _make_gather(n_sel, d):
    mesh = plsc.VectorSubcoreMesh(core_axis_name="c", subcore_axis_name="s")
    rows_per_tile = n_sel // 32

    @functools.partial(
        pl.kernel,
        out_type=jax.ShapeDtypeStruct((n_sel, d), jnp.float32),
        mesh=mesh,
        scratch_types=[
            pltpu.VMEM((rows_per_tile,), jnp.int32),
            pltpu.VMEM((rows_per_tile, d), jnp.float32),
            pltpu.SemaphoreType.DMA,
        ],
    )
    def gather_kernel(x_hbm, topi_hbm, xsel_hbm, idx_v, rows_v, sem):
        c = lax.axis_index("c")
        s = lax.axis_index("s")
        base = (s * 2 + c) * rows_per_tile
        pltpu.sync_copy(topi_hbm.at[pl.ds(base, rows_per_tile)], idx_v)
        pltpu.async_copy(x_hbm.at[idx_v], rows_v, sem).wait()
        pltpu.sync_copy(rows_v, xsel_hbm.at[pl.ds(base, rows_per_tile)])

    return gather_kernel


def _block_body(xs_ref, wt_ref, bt_ref, sw_ref, y_ref):
    y_ref[...] = (
        jnp.dot(xs_ref[...], wt_ref[...], preferred_element_type=jnp.float32)
        + bt_ref[...]
    ) * sw_ref[...]


def _block_mm(xsel, Wt, bt, selw, n_sel, d, blk):
    return pl.pallas_call(
        _block_body,
        grid=(n_sel // blk,),
        in_specs=[
            pl.BlockSpec((blk, d), lambda i: (i, 0)),
            pl.BlockSpec((d, d), lambda i: (0, 0)),
            pl.BlockSpec((1, d), lambda i: (0, 0)),
            pl.BlockSpec((blk, 1), lambda i: (i, 0)),
        ],
        out_specs=pl.BlockSpec((blk, d), lambda i: (i, 0)),
        out_shape=jax.ShapeDtypeStruct((n_sel, d), jnp.float32),
        name="mod_block_mm",
    )(xsel, Wt, bt.reshape(1, d), selw.reshape(n_sel, 1))


def _make_scatter(n_rows, d, n_sel):
    mesh = plsc.VectorSubcoreMesh(core_axis_name="c", subcore_axis_name="s")
    sel_per_tile = n_sel // 32

    @functools.partial(
        pl.kernel,
        out_type=(),
        mesh=mesh,
        scratch_types=[
            pltpu.VMEM((sel_per_tile, d), jnp.float32),
            pltpu.VMEM((sel_per_tile,), jnp.int32),
            pltpu.SemaphoreType.DMA,
        ],
    )
    def scatter_kernel(out_ref, topi_hbm, y_hbm, buf, idx_v, sem):
        c = lax.axis_index("c")
        s = lax.axis_index("s")
        o = (s * 2 + c) * sel_per_tile
        pltpu.sync_copy(topi_hbm.at[pl.ds(o, sel_per_tile)], idx_v)
        pltpu.sync_copy(y_hbm.at[pl.ds(o, sel_per_tile)], buf)
        pltpu.async_copy(buf, out_ref.at[idx_v], sem).wait()

    return scatter_kernel


def kernel(x, Wr, br, Wa, ba, Wt, bt):
    B, S, D = x.shape
    k = S // 8
    n_rows = B * S
    n_sel = B * k
    assert B == 4 and S % 16 == 0 and D % 16 == 0 and n_sel % 32 == 0

    x_flat = x.reshape(n_rows, D)
    rw, out0 = _router(x_flat, Wr, br, n_rows, D, 1024)

    topi, selw = _make_topk(B, S, k)(rw.reshape(B, S))
    xsel = _make_gather(n_sel, D)(x_flat, topi)
    y = _block_mm(xsel, Wt, bt, selw, n_sel, D, 512)
    out_ref = jax.new_ref(out0)
    _make_scatter(n_rows, D, n_sel)(out_ref, topi, y)
    return jax.freeze(out_ref).reshape(B, S, D)


# speedup vs baseline: 4.9505x; 1.0168x over previous
import functools

import jax
import jax.numpy as jnp
from jax import lax
from jax.experimental import pallas as pl
from jax.experimental.pallas import tpu as pltpu
from jax.experimental.pallas import tpu_sc as plsc


def _router_body(x_ref, wr_ref, br_ref, rw_ref, out_ref):
    xb = x_ref[...]
    out_ref[...] = xb
    rw_ref[...] = (
        jnp.dot(xb, wr_ref[...], preferred_element_type=jnp.float32)
        + br_ref[0, 0]
    )


def _router(x_flat, Wr, br, n_rows, d, blk):
    return pl.pallas_call(
        _router_body,
        grid=(n_rows // blk,),
        in_specs=[
            pl.BlockSpec((blk, d), lambda i: (i, 0)),
            pl.BlockSpec((d, 1), lambda i: (0, 0)),
            pl.BlockSpec(memory_space=pltpu.SMEM),
        ],
        out_specs=[
            pl.BlockSpec((blk, 1), lambda i: (i, 0)),
            pl.BlockSpec((blk, d), lambda i: (i, 0)),
        ],
        out_shape=[
            jax.ShapeDtypeStruct((n_rows, 1), jnp.float32),
            jax.ShapeDtypeStruct((n_rows, d), jnp.float32),
        ],
        name="mod_router",
    )(x_flat, Wr, br.reshape(1, 1))


def _make_topk(b_count, s_len, k):
    mesh = plsc.VectorSubcoreMesh(core_axis_name="c", subcore_axis_name="s")
    batches_per_core = b_count // 2
    n_vregs = s_len // 16

    @functools.partial(
        pl.kernel,
        out_type=[
            jax.ShapeDtypeStruct((b_count * k,), jnp.int32),
            jax.ShapeDtypeStruct((b_count * k,), jnp.float32),
        ],
        mesh=mesh,
        scratch_types=[
            pltpu.VMEM((s_len,), jnp.float32),
            pltpu.VMEM((s_len,), jnp.uint32),
            pltpu.VMEM((k,), jnp.int32),
            pltpu.VMEM((k,), jnp.float32),
        ],
        compiler_params=pltpu.CompilerParams(needs_layout_passes=False),
    )
    def topk_kernel(rw_hbm, topi_hbm, selw_hbm, vals_v, keys_v, ti_v, tw_v):
        c = lax.axis_index("c")
        s = lax.axis_index("s")
        b = c * batches_per_core + s

        @pl.when(s < batches_per_core)
        def _():
            pltpu.sync_copy(rw_hbm.at[b], vals_v)

            def key_body(j, _):
                base = j * 128
                for u in range(8):
                    off = base + u * 16
                    bu = plsc.bitcast(vals_v[pl.ds(off, 16)], jnp.uint32)
                    keys_v[pl.ds(off, 16)] = jnp.where(
                        bu >= jnp.uint32(0x80000000),
                        ~bu,
                        bu | jnp.uint32(0x80000000),
                    )
                return 0

            lax.fori_loop(0, n_vregs // 8, key_body, 0)

            def count_cmp(thresh, strict):
                def body(j, acc):
                    base = j * 128
                    for u in range(8):
                        kv = keys_v[pl.ds(base + u * 16, 16)]
                        hit = kv > thresh if strict else kv >= thresh
                        acc = acc + hit.astype(jnp.int32)
                    return acc

                acc = lax.fori_loop(
                    0, n_vregs // 8, body, jnp.zeros((16,), jnp.int32)
                )
                return jnp.sum(acc)

            t = jnp.uint32(0)
            for bit in range(31, -1, -1):
                cand = t | jnp.uint32(1 << bit)
                t = jnp.where(count_cmp(cand, False) >= k, cand, t)

            need = k - count_cmp(t, True)
            lane = lax.iota(jnp.int32, 16)

            def comp_body(j, carry):
                off, eqs = carry
                base = j * 16
                kv = keys_v[pl.ds(base, 16)]
                vv = vals_v[pl.ds(base, 16)]
                gt = kv > t
                eq = kv == t
                eqc = plsc.cumsum(eq.astype(jnp.int32))
                sel = jnp.logical_or(
                    gt, jnp.logical_and(eq, (eqs + eqc) <= need)
                )
                sc = plsc.cumsum(sel.astype(jnp.int32))
                dest = off + sc - 1
                gidx = base + lane + b * s_len
                plsc.store_scatter(ti_v, [dest], gidx, mask=sel)
                plsc.store_scatter(tw_v, [dest], vv, mask=sel)
                return (
                    off + jnp.sum(sel.astype(jnp.int32)),
                    eqs + jnp.sum(eq.astype(jnp.int32)),
                )

            lax.fori_loop(0, n_vregs, comp_body, (jnp.int32(0), jnp.int32(0)))

            pltpu.sync_copy(ti_v, topi_hbm.at[pl.ds(b * k, k)])
            pltpu.sync_copy(tw_v, selw_hbm.at[pl.ds(b * k, k)])

    return topk_kernel


def _make_gather(n_sel, d):
    mesh = plsc.VectorSubcoreMesh(core_axis_name="c", subcore_axis_name="s")
    rows_per_tile = n_sel // 32

    @functools.partial(
        pl.kernel,
        out_type=jax.ShapeDtypeStruct((n_sel, d), jnp.float32),
        mesh=mesh,
        scratch_types=[
            pltpu.VMEM((rows_per_tile,), jnp.int32),
            pltpu.VMEM((rows_per_tile, d), jnp.float32),
            pltpu.SemaphoreType.DMA,
        ],
    )
    def gather_kernel(x_hbm, topi_hbm, xsel_hbm, idx_v, rows_v, sem):
        c = lax.axis_index("c")
        s = lax.axis_index("s")
        base = (s * 2 + c) * rows_per_tile
        pltpu.sync_copy(topi_hbm.at[pl.ds(base, rows_per_tile)], idx_v)
        pltpu.async_copy(x_hbm.at[idx_v], rows_v, sem).wait()
        pltpu.sync_copy(rows_v, xsel_hbm.at[pl.ds(base, rows_per_tile)])

    return gather_kernel


def _block_body(xs_ref, wt_ref, bt_ref, sw_ref, y_ref):
    y_ref[...] = (
        jnp.dot(xs_ref[...], wt_ref[...], preferred_element_type=jnp.float32)
        + bt_ref[...]
    ) * sw_ref[...]


def _block_mm(xsel, Wt, bt, selw, n_sel, d, blk):
    return pl.pallas_call(
        _block_body,
        grid=(n_sel // blk,),
        in_specs=[
            pl.BlockSpec((blk, d), lambda i: (i, 0)),
            pl.BlockSpec((d, d), lambda i: (0, 0)),
            pl.BlockSpec((1, d), lambda i: (0, 0)),
            pl.BlockSpec((blk, 1), lambda i: (i, 0)),
        ],
        out_specs=pl.BlockSpec((blk, d), lambda i: (i, 0)),
        out_shape=jax.ShapeDtypeStruct((n_sel, d), jnp.float32),
        name="mod_block_mm",
    )(xsel, Wt, bt.reshape(1, d), selw.reshape(n_sel, 1))


def _make_scatter(n_rows, d, n_sel):
    mesh = plsc.VectorSubcoreMesh(core_axis_name="c", subcore_axis_name="s")
    sel_per_tile = n_sel // 32

    @functools.partial(
        pl.kernel,
        out_type=(),
        mesh=mesh,
        scratch_types=[
            pltpu.VMEM((sel_per_tile, d), jnp.float32),
            pltpu.VMEM((sel_per_tile,), jnp.int32),
            pltpu.SemaphoreType.DMA,
        ],
    )
    def scatter_kernel(out_ref, topi_hbm, y_hbm, buf, idx_v, sem):
        c = lax.axis_index("c")
        s = lax.axis_index("s")
        o = (s * 2 + c) * sel_per_tile
        pltpu.sync_copy(topi_hbm.at[pl.ds(o, sel_per_tile)], idx_v)
        pltpu.sync_copy(y_hbm.at[pl.ds(o, sel_per_tile)], buf)
        pltpu.async_copy(buf, out_ref.at[idx_v], sem).wait()

    return scatter_kernel


def kernel(x, Wr, br, Wa, ba, Wt, bt):
    B, S, D = x.shape
    k = S // 8
    n_rows = B * S
    n_sel = B * k
    assert B == 4 and S % 16 == 0 and D % 16 == 0 and n_sel % 32 == 0

    x_flat = x.reshape(n_rows, D)
    rw, out0 = _router(x_flat, Wr, br, n_rows, D, 2048)

    topi, selw = _make_topk(B, S, k)(rw.reshape(B, S))
    xsel = _make_gather(n_sel, D)(x_flat, topi)
    y = _block_mm(xsel, Wt, bt, selw, n_sel, D, 512)
    out_ref = jax.new_ref(out0)
    _make_scatter(n_rows, D, n_sel)(out_ref, topi, y)
    return jax.freeze(out_ref).reshape(B, S, D)


# speedup vs baseline: 4.9585x; 1.0016x over previous
import functools

import jax
import jax.numpy as jnp
from jax import lax
from jax.experimental import pallas as pl
from jax.experimental.pallas import tpu as pltpu
from jax.experimental.pallas import tpu_sc as plsc


def _router_body(x_ref, wr_ref, br_ref, rw_ref, out_ref):
    xb = x_ref[...]
    out_ref[...] = xb
    rw_ref[...] = (
        jnp.dot(xb, wr_ref[...], preferred_element_type=jnp.float32)
        + br_ref[0, 0]
    )


def _router(x_flat, Wr, br, n_rows, d, blk):
    return pl.pallas_call(
        _router_body,
        grid=(n_rows // blk,),
        in_specs=[
            pl.BlockSpec((blk, d), lambda i: (i, 0)),
            pl.BlockSpec((d, 1), lambda i: (0, 0)),
            pl.BlockSpec(memory_space=pltpu.SMEM),
        ],
        out_specs=[
            pl.BlockSpec((blk, 1), lambda i: (i, 0)),
            pl.BlockSpec((blk, d), lambda i: (i, 0)),
        ],
        out_shape=[
            jax.ShapeDtypeStruct((n_rows, 1), jnp.float32),
            jax.ShapeDtypeStruct((n_rows, d), jnp.float32),
        ],
        name="mod_router",
    )(x_flat, Wr, br.reshape(1, 1))


def _make_topk(b_count, s_len, k):
    mesh = plsc.VectorSubcoreMesh(core_axis_name="c", subcore_axis_name="s")
    batches_per_core = b_count // 2
    n_vregs = s_len // 16

    @functools.partial(
        pl.kernel,
        out_type=[
            jax.ShapeDtypeStruct((b_count * k,), jnp.int32),
            jax.ShapeDtypeStruct((b_count * k,), jnp.float32),
        ],
        mesh=mesh,
        scratch_types=[
            pltpu.VMEM((s_len,), jnp.float32),
            pltpu.VMEM((s_len,), jnp.uint32),
            pltpu.VMEM((k,), jnp.int32),
            pltpu.VMEM((k,), jnp.float32),
        ],
        compiler_params=pltpu.CompilerParams(needs_layout_passes=False),
    )
    def topk_kernel(rw_hbm, topi_hbm, selw_hbm, vals_v, keys_v, ti_v, tw_v):
        c = lax.axis_index("c")
        s = lax.axis_index("s")
        b = c * batches_per_core + s

        @pl.when(s < batches_per_core)
        def _():
            pltpu.sync_copy(rw_hbm.at[b], vals_v)

            def key_body(j, _):
                base = j * 128
                for u in range(8):
                    off = base + u * 16
                    bu = plsc.bitcast(vals_v[pl.ds(off, 16)], jnp.uint32)
                    keys_v[pl.ds(off, 16)] = jnp.where(
                        bu >= jnp.uint32(0x80000000),
                        ~bu,
                        bu | jnp.uint32(0x80000000),
                    )
                return 0

            lax.fori_loop(0, n_vregs // 8, key_body, 0)

            def count_cmp(thresh, strict):
                def body(j, acc):
                    base = j * 128
                    for u in range(8):
                        kv = keys_v[pl.ds(base + u * 16, 16)]
                        hit = kv > thresh if strict else kv >= thresh
                        acc = acc + hit.astype(jnp.int32)
                    return acc

                acc = lax.fori_loop(
                    0, n_vregs // 8, body, jnp.zeros((16,), jnp.int32)
                )
                return jnp.sum(acc)

            t = jnp.uint32(0)
            for bit in range(31, -1, -1):
                cand = t | jnp.uint32(1 << bit)
                t = jnp.where(count_cmp(cand, False) >= k, cand, t)

            need = k - count_cmp(t, True)
            lane = lax.iota(jnp.int32, 16)

            def comp_body(j, carry):
                off, eqs = carry
                base = j * 16
                kv = keys_v[pl.ds(base, 16)]
                vv = vals_v[pl.ds(base, 16)]
                gt = kv > t
                eq = kv == t
                eqc = plsc.cumsum(eq.astype(jnp.int32))
                sel = jnp.logical_or(
                    gt, jnp.logical_and(eq, (eqs + eqc) <= need)
                )
                sc = plsc.cumsum(sel.astype(jnp.int32))
                dest = off + sc - 1
                gidx = base + lane + b * s_len
                plsc.store_scatter(ti_v, [dest], gidx, mask=sel)
                plsc.store_scatter(tw_v, [dest], vv, mask=sel)
                return (
                    off + jnp.sum(sel.astype(jnp.int32)),
                    eqs + jnp.sum(eq.astype(jnp.int32)),
                )

            lax.fori_loop(0, n_vregs, comp_body, (jnp.int32(0), jnp.int32(0)))

            pltpu.sync_copy(ti_v, topi_hbm.at[pl.ds(b * k, k)])
            pltpu.sync_copy(tw_v, selw_hbm.at[pl.ds(b * k, k)])

    return topk_kernel


def _make_gather(n_sel, d):
    mesh = plsc.VectorSubcoreMesh(core_axis_name="c", subcore_axis_name="s")
    rows_per_tile = n_sel // 32

    @functools.partial(
        pl.kernel,
        out_type=jax.ShapeDtypeStruct((n_sel, d), jnp.float32),
        mesh=mesh,
        scratch_types=[
            pltpu.VMEM((rows_per_tile,), jnp.int32),
            pltpu.VMEM((rows_per_tile, d), jnp.float32),
            pltpu.SemaphoreType.DMA,
        ],
    )
    def gather_kernel(x_hbm, topi_hbm, xsel_hbm, idx_v, rows_v, sem):
        c = lax.axis_index("c")
        s = lax.axis_index("s")
        base = (s * 2 + c) * rows_per_tile
        pltpu.sync_copy(topi_hbm.at[pl.ds(base, rows_per_tile)], idx_v)
        pltpu.async_copy(x_hbm.at[idx_v], rows_v, sem).wait()
        pltpu.sync_copy(rows_v, xsel_hbm.at[pl.ds(base, rows_per_tile)])

    return gather_kernel


def _block_body(xs_ref, wt_ref, bt_ref, sw_ref, y_ref):
    y_ref[...] = (
        jnp.dot(
            xs_ref[...].astype(jnp.bfloat16),
            wt_ref[...].astype(jnp.bfloat16),
            preferred_element_type=jnp.float32,
        )
        + bt_ref[...]
    ) * sw_ref[...]


def _block_mm(xsel, Wt, bt, selw, n_sel, d, blk):
    return pl.pallas_call(
        _block_body,
        grid=(n_sel // blk,),
        in_specs=[
            pl.BlockSpec((blk, d), lambda i: (i, 0)),
            pl.BlockSpec((d, d), lambda i: (0, 0)),
            pl.BlockSpec((1, d), lambda i: (0, 0)),
            pl.BlockSpec((blk, 1), lambda i: (i, 0)),
        ],
        out_specs=pl.BlockSpec((blk, d), lambda i: (i, 0)),
        out_shape=jax.ShapeDtypeStruct((n_sel, d), jnp.float32),
        name="mod_block_mm",
    )(xsel, Wt, bt.reshape(1, d), selw.reshape(n_sel, 1))


def _make_scatter(n_rows, d, n_sel):
    mesh = plsc.VectorSubcoreMesh(core_axis_name="c", subcore_axis_name="s")
    sel_per_tile = n_sel // 32

    @functools.partial(
        pl.kernel,
        out_type=(),
        mesh=mesh,
        scratch_types=[
            pltpu.VMEM((sel_per_tile, d), jnp.float32),
            pltpu.VMEM((sel_per_tile,), jnp.int32),
            pltpu.SemaphoreType.DMA,
        ],
    )
    def scatter_kernel(out_ref, topi_hbm, y_hbm, buf, idx_v, sem):
        c = lax.axis_index("c")
        s = lax.axis_index("s")
        o = (s * 2 + c) * sel_per_tile
        pltpu.sync_copy(topi_hbm.at[pl.ds(o, sel_per_tile)], idx_v)
        pltpu.sync_copy(y_hbm.at[pl.ds(o, sel_per_tile)], buf)
        pltpu.async_copy(buf, out_ref.at[idx_v], sem).wait()

    return scatter_kernel


def kernel(x, Wr, br, Wa, ba, Wt, bt):
    B, S, D = x.shape
    k = S // 8
    n_rows = B * S
    n_sel = B * k
    assert B == 4 and S % 16 == 0 and D % 16 == 0 and n_sel % 32 == 0

    x_flat = x.reshape(n_rows, D)
    rw, out0 = _router(x_flat, Wr, br, n_rows, D, 2048)

    topi, selw = _make_topk(B, S, k)(rw.reshape(B, S))
    xsel = _make_gather(n_sel, D)(x_flat, topi)
    y = _block_mm(xsel, Wt, bt, selw, n_sel, D, 512)
    out_ref = jax.new_ref(out0)
    _make_scatter(n_rows, D, n_sel)(out_ref, topi, y)
    return jax.freeze(out_ref).reshape(B, S, D)


# speedup vs baseline: 5.0357x; 1.0156x over previous
import functools

import jax
import jax.numpy as jnp
from jax import lax
from jax.experimental import pallas as pl
from jax.experimental.pallas import tpu as pltpu
from jax.experimental.pallas import tpu_sc as plsc


def _router_body(x_ref, wr_ref, br_ref, rw_ref, out_ref):
    xb = x_ref[...]
    out_ref[...] = xb
    rw_ref[...] = (
        jnp.dot(xb, wr_ref[...], preferred_element_type=jnp.float32)
        + br_ref[0, 0]
    )


def _router(x_flat, Wr, br, n_rows, d, blk):
    return pl.pallas_call(
        _router_body,
        grid=(n_rows // blk,),
        in_specs=[
            pl.BlockSpec((blk, d), lambda i: (i, 0)),
            pl.BlockSpec((d, 1), lambda i: (0, 0)),
            pl.BlockSpec(memory_space=pltpu.SMEM),
        ],
        out_specs=[
            pl.BlockSpec((blk, 1), lambda i: (i, 0)),
            pl.BlockSpec((blk, d), lambda i: (i, 0)),
        ],
        out_shape=[
            jax.ShapeDtypeStruct((n_rows, 1), jnp.float32),
            jax.ShapeDtypeStruct((n_rows, d), jnp.float32),
        ],
        name="mod_router",
    )(x_flat, Wr, br.reshape(1, 1))


def _make_topk(b_count, s_len, k):
    mesh = plsc.VectorSubcoreMesh(core_axis_name="c", subcore_axis_name="s")
    batches_per_core = b_count // 2
    n_vregs = s_len // 16

    @functools.partial(
        pl.kernel,
        out_type=[
            jax.ShapeDtypeStruct((b_count * k,), jnp.int32),
            jax.ShapeDtypeStruct((b_count * k,), jnp.float32),
        ],
        mesh=mesh,
        scratch_types=[
            pltpu.VMEM((s_len,), jnp.float32),
            pltpu.VMEM((s_len,), jnp.uint32),
            pltpu.VMEM((k,), jnp.int32),
            pltpu.VMEM((k,), jnp.float32),
        ],
        compiler_params=pltpu.CompilerParams(needs_layout_passes=False),
    )
    def topk_kernel(rw_hbm, topi_hbm, selw_hbm, vals_v, keys_v, ti_v, tw_v):
        c = lax.axis_index("c")
        s = lax.axis_index("s")
        b = c * batches_per_core + s

        @pl.when(s < batches_per_core)
        def _():
            pltpu.sync_copy(rw_hbm.at[b], vals_v)

            def key_body(j, _):
                base = j * 128
                for u in range(8):
                    off = base + u * 16
                    bu = plsc.bitcast(vals_v[pl.ds(off, 16)], jnp.uint32)
                    keys_v[pl.ds(off, 16)] = jnp.where(
                        bu >= jnp.uint32(0x80000000),
                        ~bu,
                        bu | jnp.uint32(0x80000000),
                    )
                return 0

            lax.fori_loop(0, n_vregs // 8, key_body, 0)

            def count_cmp(thresh, strict):
                def body(j, acc):
                    base = j * 128
                    for u in range(8):
                        kv = keys_v[pl.ds(base + u * 16, 16)]
                        hit = kv > thresh if strict else kv >= thresh
                        acc = acc + hit.astype(jnp.int32)
                    return acc

                acc = lax.fori_loop(
                    0, n_vregs // 8, body, jnp.zeros((16,), jnp.int32)
                )
                return jnp.sum(acc)

            t = jnp.uint32(0)
            for bit in range(31, -1, -1):
                cand = t | jnp.uint32(1 << bit)
                t = jnp.where(count_cmp(cand, False) >= k, cand, t)

            need = k - count_cmp(t, True)
            lane = lax.iota(jnp.int32, 16)

            def comp_body(j, carry):
                off, eqs = carry
                base = j * 16
                kv = keys_v[pl.ds(base, 16)]
                vv = vals_v[pl.ds(base, 16)]
                gt = kv > t
                eq = kv == t
                eqc = plsc.cumsum(eq.astype(jnp.int32))
                sel = jnp.logical_or(
                    gt, jnp.logical_and(eq, (eqs + eqc) <= need)
                )
                sc = plsc.cumsum(sel.astype(jnp.int32))
                dest = off + sc - 1
                gidx = base + lane + b * s_len
                plsc.store_scatter(ti_v, [dest], gidx, mask=sel)
                plsc.store_scatter(tw_v, [dest], vv, mask=sel)
                return (
                    off + jnp.sum(sel.astype(jnp.int32)),
                    eqs + jnp.sum(eq.astype(jnp.int32)),
                )

            lax.fori_loop(0, n_vregs, comp_body, (jnp.int32(0), jnp.int32(0)))

            pltpu.sync_copy(ti_v, topi_hbm.at[pl.ds(b * k, k)])
            pltpu.sync_copy(tw_v, selw_hbm.at[pl.ds(b * k, k)])

    return topk_kernel


def _make_gather(n_sel, d):
    mesh = plsc.VectorSubcoreMesh(core_axis_name="c", subcore_axis_name="s")
    rows_per_tile = n_sel // 32

    @functools.partial(
        pl.kernel,
        out_type=jax.ShapeDtypeStruct((n_sel, d), jnp.float32),
        mesh=mesh,
        scratch_types=[
            pltpu.VMEM((rows_per_tile,), jnp.int32),
            pltpu.VMEM((rows_per_tile, d), jnp.float32),
            pltpu.SemaphoreType.DMA,
        ],
    )
    def gather_kernel(x_hbm, topi_hbm, xsel_hbm, idx_v, rows_v, sem):
        c = lax.axis_index("c")
        s = lax.axis_index("s")
        base = (s * 2 + c) * rows_per_tile
        pltpu.sync_copy(topi_hbm.at[pl.ds(base, rows_per_tile)], idx_v)
        pltpu.async_copy(x_hbm.at[idx_v], rows_v, sem).wait()
        pltpu.sync_copy(rows_v, xsel_hbm.at[pl.ds(base, rows_per_tile)])

    return gather_kernel


def _block_body(xs_ref, wt_ref, bt_ref, sw_ref, y_ref):
    y_ref[...] = (
        jnp.dot(xs_ref[...], wt_ref[...], preferred_element_type=jnp.float32)
        + bt_ref[...]
    ) * sw_ref[...]


def _block_mm(xsel, Wt, bt, selw, n_sel, d, blk):
    return pl.pallas_call(
        _block_body,
        grid=(n_sel // blk,),
        in_specs=[
            pl.BlockSpec((blk, d), lambda i: (i, 0)),
            pl.BlockSpec((d, d), lambda i: (0, 0)),
            pl.BlockSpec((1, d), lambda i: (0, 0)),
            pl.BlockSpec((blk, 1), lambda i: (i, 0)),
        ],
        out_specs=pl.BlockSpec((blk, d), lambda i: (i, 0)),
        out_shape=jax.ShapeDtypeStruct((n_sel, d), jnp.float32),
        name="mod_block_mm",
    )(xsel, Wt, bt.reshape(1, d), selw.reshape(n_sel, 1))


def _make_scatter(n_rows, d, n_sel):
    mesh = plsc.VectorSubcoreMesh(core_axis_name="c", subcore_axis_name="s")
    sel_per_tile = n_sel // 32

    @functools.partial(
        pl.kernel,
        out_type=(),
        mesh=mesh,
        scratch_types=[
            pltpu.VMEM((sel_per_tile, d), jnp.float32),
            pltpu.VMEM((sel_per_tile,), jnp.int32),
            pltpu.SemaphoreType.DMA,
        ],
    )
    def scatter_kernel(out_ref, topi_hbm, y_hbm, buf, idx_v, sem):
        c = lax.axis_index("c")
        s = lax.axis_index("s")
        o = (s * 2 + c) * sel_per_tile
        pltpu.sync_copy(topi_hbm.at[pl.ds(o, sel_per_tile)], idx_v)
        pltpu.sync_copy(y_hbm.at[pl.ds(o, sel_per_tile)], buf)
        pltpu.async_copy(buf, out_ref.at[idx_v], sem).wait()

    return scatter_kernel


def kernel(x, Wr, br, Wa, ba, Wt, bt):
    B, S, D = x.shape
    k = S // 8
    n_rows = B * S
    n_sel = B * k
    assert B == 4 and S % 16 == 0 and D % 16 == 0 and n_sel % 32 == 0

    x_flat = x.reshape(n_rows, D)
    rw, out0 = _router(x_flat, Wr, br, n_rows, D, 4096)

    topi, selw = _make_topk(B, S, k)(rw.reshape(B, S))
    xsel = _make_gather(n_sel, D)(x_flat, topi)
    y = _block_mm(xsel, Wt, bt, selw, n_sel, D, 512)
    out_ref = jax.new_ref(out0)
    _make_scatter(n_rows, D, n_sel)(out_ref, topi, y)
    return jax.freeze(out_ref).reshape(B, S, D)


# speedup vs baseline: 5.1226x; 1.0173x over previous
import functools

import jax
import jax.numpy as jnp
from jax import lax
from jax.experimental import pallas as pl
from jax.experimental.pallas import tpu as pltpu
from jax.experimental.pallas import tpu_sc as plsc


def _router_body(x_ref, wr_ref, br_ref, rw_ref, out_ref):
    xb = x_ref[...]
    out_ref[...] = xb
    rw_ref[...] = (
        jnp.dot(xb, wr_ref[...], preferred_element_type=jnp.float32)
        + br_ref[0, 0]
    )


def _router(x_flat, Wr, br, n_rows, d, blk):
    return pl.pallas_call(
        _router_body,
        grid=(n_rows // blk,),
        in_specs=[
            pl.BlockSpec((blk, d), lambda i: (i, 0)),
            pl.BlockSpec((d, 1), lambda i: (0, 0)),
            pl.BlockSpec(memory_space=pltpu.SMEM),
        ],
        out_specs=[
            pl.BlockSpec((blk, 1), lambda i: (i, 0)),
            pl.BlockSpec((blk, d), lambda i: (i, 0)),
        ],
        out_shape=[
            jax.ShapeDtypeStruct((n_rows, 1), jnp.float32),
            jax.ShapeDtypeStruct((n_rows, d), jnp.float32),
        ],
        name="mod_router",
    )(x_flat, Wr, br.reshape(1, 1))


def _make_topk(b_count, s_len, k):
    mesh = plsc.VectorSubcoreMesh(core_axis_name="c", subcore_axis_name="s")
    batches_per_core = b_count // 2
    n_vregs = s_len // 16
    U32 = jnp.uint32
    TOP = U32(0x80000000)

    @functools.partial(
        pl.kernel,
        out_type=[
            jax.ShapeDtypeStruct((b_count * k,), jnp.int32),
            jax.ShapeDtypeStruct((b_count * k,), jnp.float32),
        ],
        mesh=mesh,
        scratch_types=[
            pltpu.VMEM((s_len,), jnp.float32),
            pltpu.VMEM((s_len + 64,), jnp.int32),
            pltpu.VMEM((s_len + 64,), jnp.int32),
            pltpu.VMEM((k,), jnp.int32),
            pltpu.VMEM((k,), jnp.float32),
        ],
        compiler_params=pltpu.CompilerParams(needs_layout_passes=False),
    )
    def topk_kernel(rw_hbm, topi_hbm, selw_hbm, vals_v, keys_v, cp_v, ti_v, tw_v):
        c = lax.axis_index("c")
        s = lax.axis_index("s")
        b = c * batches_per_core + s
        lane = lax.iota(jnp.int32, 16)

        @pl.when(s < batches_per_core)
        def _():
            pltpu.sync_copy(rw_hbm.at[b], vals_v)

            def key_body(j, acc):
                base = j * 128
                for u in range(8):
                    off = base + u * 16
                    bu = plsc.bitcast(vals_v[pl.ds(off, 16)], U32)
                    ky = jnp.where(bu >= TOP, ~bu, bu | TOP)
                    keys_v[pl.ds(off, 16)] = plsc.bitcast(ky, jnp.int32)
                    acc = acc + (ky >= TOP).astype(jnp.int32)
                return acc

            acc31 = lax.fori_loop(
                0, n_vregs // 8, key_body, jnp.zeros((16,), jnp.int32)
            )
            t = jnp.where(jnp.sum(acc31) >= k, TOP, U32(0))

            def count_full(thresh):
                def body(j, acc):
                    base = j * 128
                    for u in range(8):
                        kv = plsc.bitcast(keys_v[pl.ds(base + u * 16, 16)], U32)
                        acc = acc + (kv >= thresh).astype(jnp.int32)
                    return acc

                acc = lax.fori_loop(
                    0, n_vregs // 8, body, jnp.zeros((16,), jnp.int32)
                )
                return jnp.sum(acc)

            for bit in range(30, 23, -1):
                cand = t | U32(1 << bit)
                t = jnp.where(count_full(cand) >= k, cand, t)

            def comp_body(j, off):
                base = j * 16
                kv = keys_v[pl.ds(base, 16)]
                m = plsc.bitcast(kv, U32) >= t
                dest = off + plsc.cumsum(m.astype(jnp.int32)) - 1
                plsc.store_scatter(keys_v, [dest], kv, mask=m)
                plsc.store_scatter(cp_v, [dest], base + lane, mask=m)
                return off + jnp.sum(m.astype(jnp.int32))

            n_cand = lax.fori_loop(0, n_vregs, comp_body, jnp.int32(0))
            zero16 = jnp.zeros((16,), jnp.int32)
            for p in range(4):
                plsc.store_scatter(keys_v, [n_cand + p * 16 + lane], zero16)

            trips4 = (n_cand + 63) >> 6

            def count_cand(thresh, strict):
                def body(j, acc):
                    base = j * 64
                    for u in range(4):
                        kv = plsc.bitcast(keys_v[pl.ds(base + u * 16, 16)], U32)
                        hit = kv > thresh if strict else kv >= thresh
                        acc = acc + hit.astype(jnp.int32)
                    return acc

                acc = lax.fori_loop(0, trips4, body, jnp.zeros((16,), jnp.int32))
                return jnp.sum(acc)

            for bit in range(23, -1, -1):
                cand = t | U32(1 << bit)
                t = jnp.where(count_cand(cand, False) >= k, cand, t)

            need = k - count_cand(t, True)
            trips = (n_cand + 15) >> 4

            def sel_body(j, carry):
                off, eqs = carry
                base = j * 16
                kv = plsc.bitcast(keys_v[pl.ds(base, 16)], U32)
                pv = cp_v[pl.ds(base, 16)]
                gt = kv > t
                eq = kv == t
                eqc = plsc.cumsum(eq.astype(jnp.int32))
                sel = jnp.logical_or(gt, jnp.logical_and(eq, (eqs + eqc) <= need))
                dest = off + plsc.cumsum(sel.astype(jnp.int32)) - 1
                plsc.store_scatter(ti_v, [dest], pv + b * s_len, mask=sel)
                pv_safe = jnp.where(sel, pv, 0)
                vv = plsc.load_gather(vals_v, [pv_safe])
                plsc.store_scatter(tw_v, [dest], vv, mask=sel)
                return (
                    off + jnp.sum(sel.astype(jnp.int32)),
                    eqs + jnp.sum(eq.astype(jnp.int32)),
                )

            lax.fori_loop(0, trips, sel_body, (jnp.int32(0), jnp.int32(0)))

            pltpu.sync_copy(ti_v, topi_hbm.at[pl.ds(b * k, k)])
            pltpu.sync_copy(tw_v, selw_hbm.at[pl.ds(b * k, k)])

    return topk_kernel


def _make_gather(n_sel, d):
    mesh = plsc.VectorSubcoreMesh(core_axis_name="c", subcore_axis_name="s")
    rows_per_tile = n_sel // 32

    @functools.partial(
        pl.kernel,
        out_type=jax.ShapeDtypeStruct((n_sel, d), jnp.float32),
        mesh=mesh,
        scratch_types=[
            pltpu.VMEM((rows_per_tile,), jnp.int32),
            pltpu.VMEM((rows_per_tile, d), jnp.float32),
            pltpu.SemaphoreType.DMA,
        ],
    )
    def gather_kernel(x_hbm, topi_hbm, xsel_hbm, idx_v, rows_v, sem):
        c = lax.axis_index("c")
        s = lax.axis_index("s")
        base = (s * 2 + c) * rows_per_tile
        pltpu.sync_copy(topi_hbm.at[pl.ds(base, rows_per_tile)], idx_v)
        pltpu.async_copy(x_hbm.at[idx_v], rows_v, sem).wait()
        pltpu.sync_copy(rows_v, xsel_hbm.at[pl.ds(base, rows_per_tile)])

    return gather_kernel


def _block_body(xs_ref, wt_ref, bt_ref, sw_ref, y_ref):
    y_ref[...] = (
        jnp.dot(xs_ref[...], wt_ref[...], preferred_element_type=jnp.float32)
        + bt_ref[...]
    ) * sw_ref[...]


def _block_mm(xsel, Wt, bt, selw, n_sel, d, blk):
    return pl.pallas_call(
        _block_body,
        grid=(n_sel // blk,),
        in_specs=[
            pl.BlockSpec((blk, d), lambda i: (i, 0)),
            pl.BlockSpec((d, d), lambda i: (0, 0)),
            pl.BlockSpec((1, d), lambda i: (0, 0)),
            pl.BlockSpec((blk, 1), lambda i: (i, 0)),
        ],
        out_specs=pl.BlockSpec((blk, d), lambda i: (i, 0)),
        out_shape=jax.ShapeDtypeStruct((n_sel, d), jnp.float32),
        name="mod_block_mm",
    )(xsel, Wt, bt.reshape(1, d), selw.reshape(n_sel, 1))


def _make_scatter(n_rows, d, n_sel):
    mesh = plsc.VectorSubcoreMesh(core_axis_name="c", subcore_axis_name="s")
    sel_per_tile = n_sel // 32

    @functools.partial(
        pl.kernel,
        out_type=(),
        mesh=mesh,
        scratch_types=[
            pltpu.VMEM((sel_per_tile, d), jnp.float32),
            pltpu.VMEM((sel_per_tile,), jnp.int32),
            pltpu.SemaphoreType.DMA,
        ],
    )
    def scatter_kernel(out_ref, topi_hbm, y_hbm, buf, idx_v, sem):
        c = lax.axis_index("c")
        s = lax.axis_index("s")
        o = (s * 2 + c) * sel_per_tile
        pltpu.sync_copy(topi_hbm.at[pl.ds(o, sel_per_tile)], idx_v)
        pltpu.sync_copy(y_hbm.at[pl.ds(o, sel_per_tile)], buf)
        pltpu.async_copy(buf, out_ref.at[idx_v], sem).wait()

    return scatter_kernel


def kernel(x, Wr, br, Wa, ba, Wt, bt):
    B, S, D = x.shape
    k = S // 8
    n_rows = B * S
    n_sel = B * k
    assert B == 4 and S % 16 == 0 and D % 16 == 0 and n_sel % 32 == 0

    x_flat = x.reshape(n_rows, D)
    rw, out0 = _router(x_flat, Wr, br, n_rows, D, 4096)

    topi, selw = _make_topk(B, S, k)(rw.reshape(B, S))
    xsel = _make_gather(n_sel, D)(x_flat, topi)
    y = _block_mm(xsel, Wt, bt, selw, n_sel, D, 512)
    out_ref = jax.new_ref(out0)
    _make_scatter(n_rows, D, n_sel)(out_ref, topi, y)
    return jax.freeze(out_ref).reshape(B, S, D)


# speedup vs baseline: 5.1319x; 1.0018x over previous
import functools

import jax
import jax.numpy as jnp
from jax import lax
from jax.experimental import pallas as pl
from jax.experimental.pallas import tpu as pltpu
from jax.experimental.pallas import tpu_sc as plsc


def _router_body(x_ref, wr_ref, br_ref, rw_ref, out_ref):
    xb = x_ref[...]
    out_ref[...] = xb
    rw_ref[...] = (
        jnp.dot(xb, wr_ref[...], preferred_element_type=jnp.float32)
        + br_ref[0, 0]
    )


def _router(x_flat, Wr, br, n_rows, d, blk):
    return pl.pallas_call(
        _router_body,
        grid=(n_rows // blk,),
        in_specs=[
            pl.BlockSpec((blk, d), lambda i: (i, 0)),
            pl.BlockSpec((d, 1), lambda i: (0, 0)),
            pl.BlockSpec(memory_space=pltpu.SMEM),
        ],
        out_specs=[
            pl.BlockSpec((blk, 1), lambda i: (i, 0)),
            pl.BlockSpec((blk, d), lambda i: (i, 0)),
        ],
        out_shape=[
            jax.ShapeDtypeStruct((n_rows, 1), jnp.float32),
            jax.ShapeDtypeStruct((n_rows, d), jnp.float32),
        ],
        name="mod_router",
    )(x_flat, Wr, br.reshape(1, 1))


def _make_topk(b_count, s_len, k, d):
    mesh = plsc.VectorSubcoreMesh(core_axis_name="c", subcore_axis_name="s")
    batches_per_core = b_count // 2
    n_sel = b_count * k
    sel_per_core = n_sel // 2
    n_vregs = s_len // 16
    chunk = 64
    U32 = jnp.uint32
    TOP = U32(0x80000000)

    @functools.partial(
        pl.kernel,
        out_type=[
            jax.ShapeDtypeStruct((n_sel,), jnp.int32),
            jax.ShapeDtypeStruct((n_sel,), jnp.float32),
            jax.ShapeDtypeStruct((n_sel, d), jnp.float32),
        ],
        mesh=mesh,
        scratch_types=[
            pltpu.VMEM((s_len,), jnp.float32),
            pltpu.VMEM((s_len + 64,), jnp.int32),
            pltpu.VMEM((s_len + 64,), jnp.int32),
            pltpu.VMEM((k,), jnp.int32),
            pltpu.VMEM((k,), jnp.float32),
            pltpu.VMEM((chunk,), jnp.int32),
            pltpu.VMEM((chunk,), jnp.int32),
            pltpu.VMEM((chunk, d), jnp.float32),
            pltpu.VMEM((chunk, d), jnp.float32),
            pltpu.VMEM_SHARED((sel_per_core,), jnp.int32),
            pltpu.SemaphoreType.DMA,
        ],
        compiler_params=pltpu.CompilerParams(needs_layout_passes=False),
    )
    def topk_kernel(
        rw_hbm, x_hbm, topi_hbm, selw_hbm, xsel_hbm,
        vals_v, keys_v, cp_v, ti_v, tw_v,
        idx0_v, idx1_v, rows0_v, rows1_v, shared_idx, sem,
    ):
        c = lax.axis_index("c")
        s = lax.axis_index("s")
        b = c * batches_per_core + s
        lane = lax.iota(jnp.int32, 16)

        @pl.when(s < batches_per_core)
        def _():
            pltpu.sync_copy(rw_hbm.at[b], vals_v)

            def key_body(j, acc):
                base = j * 128
                for u in range(8):
                    off = base + u * 16
                    bu = plsc.bitcast(vals_v[pl.ds(off, 16)], U32)
                    ky = jnp.where(bu >= TOP, ~bu, bu | TOP)
                    keys_v[pl.ds(off, 16)] = plsc.bitcast(ky, jnp.int32)
                    acc = acc + (ky >= TOP).astype(jnp.int32)
                return acc

            acc31 = lax.fori_loop(
                0, n_vregs // 8, key_body, jnp.zeros((16,), jnp.int32)
            )
            t = jnp.where(jnp.sum(acc31) >= k, TOP, U32(0))

            def count_full(thresh):
                def body(j, acc):
                    base = j * 128
                    for u in range(8):
                        kv = plsc.bitcast(keys_v[pl.ds(base + u * 16, 16)], U32)
                        acc = acc + (kv >= thresh).astype(jnp.int32)
                    return acc

                acc = lax.fori_loop(
                    0, n_vregs // 8, body, jnp.zeros((16,), jnp.int32)
                )
                return jnp.sum(acc)

            for bit in range(30, 23, -1):
                cand = t | U32(1 << bit)
                t = jnp.where(count_full(cand) >= k, cand, t)

            def comp_body(j, off):
                base = j * 16
                kv = keys_v[pl.ds(base, 16)]
                m = plsc.bitcast(kv, U32) >= t
                dest = off + plsc.cumsum(m.astype(jnp.int32)) - 1
                plsc.store_scatter(keys_v, [dest], kv, mask=m)
                plsc.store_scatter(cp_v, [dest], base + lane, mask=m)
                return off + jnp.sum(m.astype(jnp.int32))

            n_cand = lax.fori_loop(0, n_vregs, comp_body, jnp.int32(0))
            zero16 = jnp.zeros((16,), jnp.int32)
            for p in range(4):
                plsc.store_scatter(keys_v, [n_cand + p * 16 + lane], zero16)

            trips4 = (n_cand + 63) >> 6

            def count_cand(thresh, strict):
                def body(j, acc):
                    base = j * 64
                    for u in range(4):
                        kv = plsc.bitcast(keys_v[pl.ds(base + u * 16, 16)], U32)
                        hit = kv > thresh if strict else kv >= thresh
                        acc = acc + hit.astype(jnp.int32)
                    return acc

                acc = lax.fori_loop(0, trips4, body, jnp.zeros((16,), jnp.int32))
                return jnp.sum(acc)

            for bit in range(23, -1, -1):
                cand = t | U32(1 << bit)
                t = jnp.where(count_cand(cand, False) >= k, cand, t)

            need = k - count_cand(t, True)
            trips = (n_cand + 15) >> 4

            def sel_body(j, carry):
                off, eqs = carry
                base = j * 16
                kv = plsc.bitcast(keys_v[pl.ds(base, 16)], U32)
                pv = cp_v[pl.ds(base, 16)]
                gt = kv > t
                eq = kv == t
                eqc = plsc.cumsum(eq.astype(jnp.int32))
                sel = jnp.logical_or(gt, jnp.logical_and(eq, (eqs + eqc) <= need))
                dest = off + plsc.cumsum(sel.astype(jnp.int32)) - 1
                plsc.store_scatter(ti_v, [dest], pv + b * s_len, mask=sel)
                pv_safe = jnp.where(sel, pv, 0)
                vv = plsc.load_gather(vals_v, [pv_safe])
                plsc.store_scatter(tw_v, [dest], vv, mask=sel)
                return (
                    off + jnp.sum(sel.astype(jnp.int32)),
                    eqs + jnp.sum(eq.astype(jnp.int32)),
                )

            lax.fori_loop(0, trips, sel_body, (jnp.int32(0), jnp.int32(0)))

            pltpu.sync_copy(ti_v, topi_hbm.at[pl.ds(b * k, k)])
            pltpu.sync_copy(tw_v, selw_hbm.at[pl.ds(b * k, k)])
            pltpu.sync_copy(ti_v, shared_idx.at[pl.ds(s * k, k)])

        plsc.subcore_barrier()
        o0 = s * 2 * chunk
        o1 = o0 + chunk
        gbase = c * sel_per_core
        pltpu.sync_copy(shared_idx.at[pl.ds(o0, chunk)], idx0_v)
        pltpu.async_copy(x_hbm.at[idx0_v], rows0_v, sem).wait()
        pltpu.sync_copy(rows0_v, xsel_hbm.at[pl.ds(gbase + o0, chunk)])
        pltpu.sync_copy(shared_idx.at[pl.ds(o1, chunk)], idx1_v)
        pltpu.async_copy(x_hbm.at[idx1_v], rows1_v, sem).wait()
        pltpu.sync_copy(rows1_v, xsel_hbm.at[pl.ds(gbase + o1, chunk)])

    return topk_kernel


def _block_body(xs_ref, wt_ref, bt_ref, sw_ref, y_ref):
    y_ref[...] = (
        jnp.dot(xs_ref[...], wt_ref[...], preferred_element_type=jnp.float32)
        + bt_ref[...]
    ) * sw_ref[...]


def _block_mm(xsel, Wt, bt, selw, n_sel, d, blk):
    return pl.pallas_call(
        _block_body,
        grid=(n_sel // blk,),
        in_specs=[
            pl.BlockSpec((blk, d), lambda i: (i, 0)),
            pl.BlockSpec((d, d), lambda i: (0, 0)),
            pl.BlockSpec((1, d), lambda i: (0, 0)),
            pl.BlockSpec((blk, 1), lambda i: (i, 0)),
        ],
        out_specs=pl.BlockSpec((blk, d), lambda i: (i, 0)),
        out_shape=jax.ShapeDtypeStruct((n_sel, d), jnp.float32),
        name="mod_block_mm",
    )(xsel, Wt, bt.reshape(1, d), selw.reshape(n_sel, 1))


def _make_scatter(n_rows, d, n_sel):
    mesh = plsc.VectorSubcoreMesh(core_axis_name="c", subcore_axis_name="s")
    sel_per_tile = n_sel // 32

    @functools.partial(
        pl.kernel,
        out_type=(),
        mesh=mesh,
        scratch_types=[
            pltpu.VMEM((sel_per_tile, d), jnp.float32),
            pltpu.VMEM((sel_per_tile,), jnp.int32),
            pltpu.SemaphoreType.DMA,
        ],
    )
    def scatter_kernel(out_ref, topi_hbm, y_hbm, buf, idx_v, sem):
        c = lax.axis_index("c")
        s = lax.axis_index("s")
        o = (s * 2 + c) * sel_per_tile
        pltpu.sync_copy(topi_hbm.at[pl.ds(o, sel_per_tile)], idx_v)
        pltpu.sync_copy(y_hbm.at[pl.ds(o, sel_per_tile)], buf)
        pltpu.async_copy(buf, out_ref.at[idx_v], sem).wait()

    return scatter_kernel


def kernel(x, Wr, br, Wa, ba, Wt, bt):
    B, S, D = x.shape
    k = S // 8
    n_rows = B * S
    n_sel = B * k
    assert B == 4 and S % 16 == 0 and D % 16 == 0 and n_sel % 32 == 0

    x_flat = x.reshape(n_rows, D)
    rw, out0 = _router(x_flat, Wr, br, n_rows, D, 4096)

    topi, selw, xsel = _make_topk(B, S, k, D)(rw.reshape(B, S), x_flat)
    y = _block_mm(xsel, Wt, bt, selw, n_sel, D, 512)
    out_ref = jax.new_ref(out0)
    _make_scatter(n_rows, D, n_sel)(out_ref, topi, y)
    return jax.freeze(out_ref).reshape(B, S, D)
